# baseline (speedup 1.0000x reference)
import functools

import jax
import jax.numpy as jnp
from jax import lax
from jax.experimental import pallas as pl
from jax.experimental.pallas import tpu as pltpu
from jax.experimental.pallas import tpu_sc as plsc

NV = 2048
NF = 100000
NC, NS = 2, 16
NW = NC * NS
FPW = 3200
FPAD = NW * FPW
CF = 800
NCF = FPW // CF
NSLOT = 3
E = NSLOT * FPAD
CE = 12800
NCH = E // CE
NDIA = 5 * NV
ROWS = 32
ACC = ROWS * NV
NRANGE = NV // ROWS
BM = 512

_f32 = jnp.float32
_i32 = jnp.int32
_u32 = jnp.uint32


def _rsqrt(s):
    yi = jnp.int32(0x5F3759DF) - lax.shift_right_logical(
        plsc.bitcast(s, _i32), 1)
    y = plsc.bitcast(yi, _f32)
    for _ in range(3):
        y = y * (1.5 - 0.5 * s * y * y)
    return y


def _build_body(vrt_hbm, fc_hbm, idx_hbm, val_hbm, dia_hbm,
                vx, vy, vz, f0b, f1b, f2b, idxst, valst, diacc, sem):
    wid = lax.axis_index("c") * NS + lax.axis_index("s")
    fbase = wid * FPW
    zeros = jnp.zeros((16,), _f32)

    @plsc.parallel_loop(0, NDIA // 16, unroll=8)
    def dz(i):
        diacc[pl.ds(i * 16, 16)] = zeros

    pltpu.sync_copy(vrt_hbm.at[pl.ds(0, NV)], vx)
    pltpu.sync_copy(vrt_hbm.at[pl.ds(NV, NV)], vy)
    pltpu.sync_copy(vrt_hbm.at[pl.ds(2 * NV, NV)], vz)
    pltpu.sync_copy(fc_hbm.at[pl.ds(fbase, FPW)], f0b)
    pltpu.sync_copy(fc_hbm.at[pl.ds(FPAD + fbase, FPW)], f1b)
    pltpu.sync_copy(fc_hbm.at[pl.ds(2 * FPAD + fbase, FPW)], f2b)
    pending = []
    for cf in range(NCF):
        for cp in pending:
            cp.wait()
        pending = []

        def jbody(j, _, cf=cf):
            pos = cf * CF + j * 16
            fa = f0b[pl.ds(pos, 16)]
            fb = f1b[pl.ds(pos, 16)]
            fc = f2b[pl.ds(pos, 16)]
            xa = plsc.load_gather(vx, [fa])
            ya = plsc.load_gather(vy, [fa])
            za = plsc.load_gather(vz, [fa])
            xb = plsc.load_gather(vx, [fb])
            yb = plsc.load_gather(vy, [fb])
            zb = plsc.load_gather(vz, [fb])
            xc = plsc.load_gather(vx, [fc])
            yc = plsc.load_gather(vy, [fc])
            zc = plsc.load_gather(vz, [fc])
            e1x, e1y, e1z = xb - xa, yb - ya, zb - za
            e2x, e2y, e2z = xc - xa, yc - ya, zc - za
            cxv = e1y * e2z - e1z * e2y
            cyv = e1z * e2x - e1x * e2z
            czv = e1x * e2y - e1y * e2x
            s = cxv * cxv + cyv * cyv + czv * czv
            inv = _rsqrt(s)
            n1 = e1x * e1x + e1y * e1y + e1z * e1z
            n2 = e2x * e2x + e2y * e2y + e2z * e2z
            d = e1x * e2x + e1y * e2y + e1z * e2z
            valid = (fbase + pos + lax.iota(_i32, 16)) < NF
            c0 = jnp.where(valid, d * inv, 0.0)
            c1 = jnp.where(valid, (n1 - d) * inv, 0.0)
            c2 = jnp.where(valid, (n2 - d) * inv, 0.0)
            area = jnp.where(valid, 0.5 * s * inv, 0.0)
            fz = jnp.where(valid, 1, 0)
            idxs = [fa * NV + fb, fb * NV + fc, fc * NV + fa]
            for si in range(NSLOT):
                idxst[pl.ds(si * CF + j * 16, 16)] = idxs[si] * fz
            h0, h1, h2 = 0.5 * c0, 0.5 * c1, 0.5 * c2
            sv = [-h2, -h0, -h1]
            dv0 = [h1 + h2, h0 + h2, h0 + h1]
            dg = area * (1.0 / 6.0)
            og = area * (1.0 / 12.0)
            mv = [og, og, og]
            dv1 = [dg, dg, dg]
            for si in range(NSLOT):
                valst[pl.ds(si * CF + j * 16, 16)] = sv[si]
                valst[pl.ds((NSLOT + si) * CF + j * 16, 16)] = mv[si]
            a30 = area * (1.0 / 30.0)
            a60 = area * (1.0 / 60.0)
            dpos = []
            for m, (p, q, r) in enumerate(
                    ((xa, xb, xc), (ya, yb, yc), (za, zb, zc)), start=2):
                p00 = a30 * (3.0 * p + q + r)
                p11 = a30 * (p + 3.0 * q + r)
                p22 = a30 * (p + q + 3.0 * r)
                p01 = a60 * (2.0 * p + 2.0 * q + r)
                p12 = a60 * (p + 2.0 * q + 2.0 * r)
                p02 = a60 * (2.0 * p + q + 2.0 * r)
                pv = [p01, p12, p02]
                dpos.append([p00, p11, p22])
                for si in range(NSLOT):
                    valst[pl.ds((m * NSLOT + si) * CF + j * 16, 16)] = pv[si]
            for m, dvals in enumerate([dv0, dv1] + dpos):
                for fv, dval in zip((fa, fb, fc), dvals):
                    plsc.addupdate_scatter(
                        diacc, [fv + m * NV], dval, mask=valid)
            return 0

        lax.fori_loop(0, CF // 16, jbody, 0)
        for si in range(NSLOT):
            dst = idx_hbm.at[pl.ds(si * FPAD + fbase + cf * CF, CF)]
            pending.append(pltpu.async_copy(idxst.at[pl.ds(si * CF, CF)], dst, sem))
        for m in range(5):
            for si in range(NSLOT):
                dst = val_hbm.at[pl.ds(m * E + si * FPAD + fbase + cf * CF, CF)]
                pending.append(pltpu.async_copy(
                    valst.at[pl.ds((m * NSLOT + si) * CF, CF)], dst, sem))
    for cp in pending:
        cp.wait()
    pltpu.sync_copy(diacc, dia_hbm.at[pl.ds(wid * NDIA, NDIA)])


def _scatter_body(idx_hbm, val_hbm, out_hbm,
                  acc, idxb0, valb0, idxb1, valb1, sem0, sem1):
    wid = lax.axis_index("c") * NS + lax.axis_index("s")
    zeros = jnp.zeros((16,), _f32)
    accu = jnp.uint32(ACC)

    def _start(c, bi, bv, sem, m):
        cs = lax.rem(c + wid * 2, NCH)
        pltpu.async_copy(idx_hbm.at[pl.ds(cs * CE, CE)], bi, sem)
        pltpu.async_copy(val_hbm.at[pl.ds(m * E + cs * CE, CE)], bv, sem)

    def _wait(bi, bv, sem):
        pltpu.make_async_copy(idx_hbm.at[pl.ds(0, CE)], bi, sem).wait()
        pltpu.make_async_copy(val_hbm.at[pl.ds(0, CE)], bv, sem).wait()

    def _scan(bi, bv, lo):
        @plsc.parallel_loop(0, CE // 16, unroll=12)
        def ibody(i):
            off = pl.ds(i * 16, 16)
            iv = bi[off]
            vv = bv[off]
            loc = iv - lo
            msk = plsc.bitcast(loc, _u32) < accu
            plsc.addupdate_scatter(acc, [loc], vv, mask=msk)

    for m in range(5):
        for h in range(2):
            g = wid * 2 + h
            lo = g * ACC
            _start(0, idxb0, valb0, sem0, m)

            @plsc.parallel_loop(0, ACC // 16, unroll=8)
            def zbody(i):
                acc[pl.ds(i * 16, 16)] = zeros

            def cbody(cc, _, m=m, lo=lo):
                c1 = 2 * cc + 1
                cn = jnp.minimum(2 * cc + 2, NCH - 1)
                _wait(idxb0, valb0, sem0)
                _start(c1, idxb1, valb1, sem1, m)
                _scan(idxb0, valb0, lo)
                _start(cn, idxb0, valb0, sem0, m)
                _wait(idxb1, valb1, sem1)
                _scan(idxb1, valb1, lo)
                return 0

            lax.fori_loop(0, NCH // 2, cbody, 0)
            _wait(idxb0, valb0, sem0)
            pltpu.sync_copy(acc, out_hbm.at[pl.ds((m * NRANGE + g) * ACC, ACC)])


def _sym_body(l_ref, lt_ref, d_ref, o_ref):
    i = pl.program_id(1)
    j = pl.program_id(2)
    a = l_ref[0]
    bt = lt_ref[0].T
    dsum = jnp.sum(d_ref[0], axis=0)
    rows = lax.broadcasted_iota(_i32, (BM, BM), 0) + i * BM
    cols = lax.broadcasted_iota(_i32, (BM, BM), 1) + j * BM
    o_ref[0] = a + bt + jnp.where(rows == cols, dsum[None, :], 0.0)


_mesh = plsc.VectorSubcoreMesh(core_axis_name="c", subcore_axis_name="s")

_build = pl.kernel(
    _build_body,
    out_type=(jax.ShapeDtypeStruct((E,), _i32),
              jax.ShapeDtypeStruct((5 * E,), _f32),
              jax.ShapeDtypeStruct((NW * NDIA,), _f32)),
    mesh=_mesh,
    compiler_params=pltpu.CompilerParams(needs_layout_passes=False),
    scratch_types=[
        pltpu.VMEM((NV,), _f32),
        pltpu.VMEM((NV,), _f32),
        pltpu.VMEM((NV,), _f32),
        pltpu.VMEM((FPW,), _i32),
        pltpu.VMEM((FPW,), _i32),
        pltpu.VMEM((FPW,), _i32),
        pltpu.VMEM((NSLOT * CF,), _i32),
        pltpu.VMEM((5 * NSLOT * CF,), _f32),
        pltpu.VMEM((NDIA,), _f32),
        pltpu.SemaphoreType.DMA,
    ],
)

_scatter = pl.kernel(
    _scatter_body,
    out_type=jax.ShapeDtypeStruct((5 * NRANGE * ACC,), _f32),
    mesh=_mesh,
    compiler_params=pltpu.CompilerParams(needs_layout_passes=False),
    scratch_types=[
        pltpu.VMEM((ACC,), _f32),
        pltpu.VMEM((CE,), _i32),
        pltpu.VMEM((CE,), _f32),
        pltpu.VMEM((CE,), _i32),
        pltpu.VMEM((CE,), _f32),
        pltpu.SemaphoreType.DMA,
        pltpu.SemaphoreType.DMA,
    ],
)

_sym = pl.pallas_call(
    _sym_body,
    grid=(5, NV // BM, NV // BM),
    in_specs=[
        pl.BlockSpec((1, BM, BM), lambda m, i, j: (m, i, j)),
        pl.BlockSpec((1, BM, BM), lambda m, i, j: (m, j, i)),
        pl.BlockSpec((1, NW, BM), lambda m, i, j: (m, 0, j)),
    ],
    out_specs=pl.BlockSpec((1, BM, BM), lambda m, i, j: (m, i, j)),
    out_shape=jax.ShapeDtypeStruct((5, NV, NV), _f32),
)


def kernel(vertices, faces):
    vrt = vertices.T.reshape(-1)
    fcs = jnp.pad(faces, ((0, FPAD - NF), (0, 0))).T.reshape(-1)
    idx_e, vals_e, dia = _build(vrt, fcs)
    lower = _scatter(idx_e, vals_e).reshape(5, NV, NV)
    return _sym(lower, lower, dia.reshape(NW, 5, NV).transpose(1, 0, 2))


# speedup vs baseline: 1.0111x; 1.0111x over previous
import functools

import jax
import jax.numpy as jnp
from jax import lax
from jax.experimental import pallas as pl
from jax.experimental.pallas import tpu as pltpu
from jax.experimental.pallas import tpu_sc as plsc

NV = 2048
NF = 100000
NC, NS = 2, 16
NW = NC * NS
FPW = 3200
FPAD = NW * FPW
CF = 800
NCF = FPW // CF
NSLOT = 3
E = NSLOT * FPAD
CE = 12800
NCH = E // CE
NDIA = 5 * NV
ROWS = 32
ACC = ROWS * NV
NRANGE = NV // ROWS
BM = 512

_f32 = jnp.float32
_i32 = jnp.int32
_u32 = jnp.uint32


def _rsqrt(s):
    yi = jnp.int32(0x5F3759DF) - lax.shift_right_logical(
        plsc.bitcast(s, _i32), 1)
    y = plsc.bitcast(yi, _f32)
    for _ in range(3):
        y = y * (1.5 - 0.5 * s * y * y)
    return y


def _build_body(vrt_hbm, fc_hbm, idx_hbm, val_hbm, dia_hbm,
                vx, vy, vz, f0b, f1b, f2b, idxst, valst, diacc, sem):
    wid = lax.axis_index("c") * NS + lax.axis_index("s")
    fbase = wid * FPW
    zeros = jnp.zeros((16,), _f32)

    @plsc.parallel_loop(0, NDIA // 16, unroll=8)
    def dz(i):
        diacc[pl.ds(i * 16, 16)] = zeros

    pltpu.sync_copy(vrt_hbm.at[pl.ds(0, NV)], vx)
    pltpu.sync_copy(vrt_hbm.at[pl.ds(NV, NV)], vy)
    pltpu.sync_copy(vrt_hbm.at[pl.ds(2 * NV, NV)], vz)
    pltpu.sync_copy(fc_hbm.at[pl.ds(fbase, FPW)], f0b)
    pltpu.sync_copy(fc_hbm.at[pl.ds(FPAD + fbase, FPW)], f1b)
    pltpu.sync_copy(fc_hbm.at[pl.ds(2 * FPAD + fbase, FPW)], f2b)
    pending = []
    for cf in range(NCF):
        for cp in pending:
            cp.wait()
        pending = []

        def jbody(j, _, cf=cf):
            pos = cf * CF + j * 16
            fa = f0b[pl.ds(pos, 16)]
            fb = f1b[pl.ds(pos, 16)]
            fc = f2b[pl.ds(pos, 16)]
            xa = plsc.load_gather(vx, [fa])
            ya = plsc.load_gather(vy, [fa])
            za = plsc.load_gather(vz, [fa])
            xb = plsc.load_gather(vx, [fb])
            yb = plsc.load_gather(vy, [fb])
            zb = plsc.load_gather(vz, [fb])
            xc = plsc.load_gather(vx, [fc])
            yc = plsc.load_gather(vy, [fc])
            zc = plsc.load_gather(vz, [fc])
            e1x, e1y, e1z = xb - xa, yb - ya, zb - za
            e2x, e2y, e2z = xc - xa, yc - ya, zc - za
            cxv = e1y * e2z - e1z * e2y
            cyv = e1z * e2x - e1x * e2z
            czv = e1x * e2y - e1y * e2x
            s = cxv * cxv + cyv * cyv + czv * czv
            inv = _rsqrt(s)
            n1 = e1x * e1x + e1y * e1y + e1z * e1z
            n2 = e2x * e2x + e2y * e2y + e2z * e2z
            d = e1x * e2x + e1y * e2y + e1z * e2z
            valid = (fbase + pos + lax.iota(_i32, 16)) < NF
            c0 = jnp.where(valid, d * inv, 0.0)
            c1 = jnp.where(valid, (n1 - d) * inv, 0.0)
            c2 = jnp.where(valid, (n2 - d) * inv, 0.0)
            area = jnp.where(valid, 0.5 * s * inv, 0.0)
            fz = jnp.where(valid, 1, 0)
            idxs = [fa * NV + fb, fb * NV + fc, fc * NV + fa]
            for si in range(NSLOT):
                idxst[pl.ds(si * CF + j * 16, 16)] = idxs[si] * fz
            h0, h1, h2 = 0.5 * c0, 0.5 * c1, 0.5 * c2
            sv = [-h2, -h0, -h1]
            dv0 = [h1 + h2, h0 + h2, h0 + h1]
            dg = area * (1.0 / 6.0)
            og = area * (1.0 / 12.0)
            mv = [og, og, og]
            dv1 = [dg, dg, dg]
            for si in range(NSLOT):
                valst[pl.ds(si * CF + j * 16, 16)] = sv[si]
                valst[pl.ds((NSLOT + si) * CF + j * 16, 16)] = mv[si]
            a30 = area * (1.0 / 30.0)
            a60 = area * (1.0 / 60.0)
            dpos = []
            for m, (p, q, r) in enumerate(
                    ((xa, xb, xc), (ya, yb, yc), (za, zb, zc)), start=2):
                p00 = a30 * (3.0 * p + q + r)
                p11 = a30 * (p + 3.0 * q + r)
                p22 = a30 * (p + q + 3.0 * r)
                p01 = a60 * (2.0 * p + 2.0 * q + r)
                p12 = a60 * (p + 2.0 * q + 2.0 * r)
                p02 = a60 * (2.0 * p + q + 2.0 * r)
                pv = [p01, p12, p02]
                dpos.append([p00, p11, p22])
                for si in range(NSLOT):
                    valst[pl.ds((m * NSLOT + si) * CF + j * 16, 16)] = pv[si]
            for m, dvals in enumerate([dv0, dv1] + dpos):
                for fv, dval in zip((fa, fb, fc), dvals):
                    plsc.addupdate_scatter(
                        diacc, [fv + m * NV], dval, mask=valid)
            return 0

        lax.fori_loop(0, CF // 16, jbody, 0)
        for si in range(NSLOT):
            dst = idx_hbm.at[pl.ds(si * FPAD + fbase + cf * CF, CF)]
            pending.append(pltpu.async_copy(idxst.at[pl.ds(si * CF, CF)], dst, sem))
        for m in range(5):
            for si in range(NSLOT):
                dst = val_hbm.at[pl.ds(m * E + si * FPAD + fbase + cf * CF, CF)]
                pending.append(pltpu.async_copy(
                    valst.at[pl.ds((m * NSLOT + si) * CF, CF)], dst, sem))
    for cp in pending:
        cp.wait()
    pltpu.sync_copy(diacc, dia_hbm.at[pl.ds(wid * NDIA, NDIA)])


def _scatter_body(idx_hbm, val_hbm, out_hbm,
                  acc, idxb0, valb0, idxb1, valb1, sem0, sem1):
    wid = lax.axis_index("c") * NS + lax.axis_index("s")
    zeros = jnp.zeros((16,), _f32)
    accu = jnp.uint32(ACC)

    def _start(c, bi, bv, sem, m):
        cs = lax.rem(c + wid * 2, NCH)
        pltpu.async_copy(idx_hbm.at[pl.ds(cs * CE, CE)], bi, sem)
        pltpu.async_copy(val_hbm.at[pl.ds(m * E + cs * CE, CE)], bv, sem)

    def _wait(bi, bv, sem):
        pltpu.make_async_copy(idx_hbm.at[pl.ds(0, CE)], bi, sem).wait()
        pltpu.make_async_copy(val_hbm.at[pl.ds(0, CE)], bv, sem).wait()

    def _scan(bi, bv, lo):
        @plsc.parallel_loop(0, CE // 16, unroll=8)
        def ibody(i):
            off = pl.ds(i * 16, 16)
            iv = bi[off]
            vv = bv[off]
            loc = iv - lo
            msk = plsc.bitcast(loc, _u32) < accu
            plsc.addupdate_scatter(acc, [loc], vv, mask=msk)

    for m in range(5):
        for h in range(2):
            g = wid * 2 + h
            lo = g * ACC
            _start(0, idxb0, valb0, sem0, m)

            @plsc.parallel_loop(0, ACC // 16, unroll=8)
            def zbody(i):
                acc[pl.ds(i * 16, 16)] = zeros

            def cbody(cc, _, m=m, lo=lo):
                c1 = 2 * cc + 1
                cn = jnp.minimum(2 * cc + 2, NCH - 1)
                _wait(idxb0, valb0, sem0)
                _start(c1, idxb1, valb1, sem1, m)
                _scan(idxb0, valb0, lo)
                _start(cn, idxb0, valb0, sem0, m)
                _wait(idxb1, valb1, sem1)
                _scan(idxb1, valb1, lo)
                return 0

            lax.fori_loop(0, NCH // 2, cbody, 0)
            _wait(idxb0, valb0, sem0)
            pltpu.sync_copy(acc, out_hbm.at[pl.ds((m * NRANGE + g) * ACC, ACC)])


def _sym_body(l_ref, lt_ref, d_ref, o_ref):
    i = pl.program_id(1)
    j = pl.program_id(2)
    a = l_ref[0]
    bt = lt_ref[0].T
    dsum = jnp.sum(d_ref[0], axis=0)
    rows = lax.broadcasted_iota(_i32, (BM, BM), 0) + i * BM
    cols = lax.broadcasted_iota(_i32, (BM, BM), 1) + j * BM
    o_ref[0] = a + bt + jnp.where(rows == cols, dsum[None, :], 0.0)


_mesh = plsc.VectorSubcoreMesh(core_axis_name="c", subcore_axis_name="s")

_build = pl.kernel(
    _build_body,
    out_type=(jax.ShapeDtypeStruct((E,), _i32),
              jax.ShapeDtypeStruct((5 * E,), _f32),
              jax.ShapeDtypeStruct((NW * NDIA,), _f32)),
    mesh=_mesh,
    compiler_params=pltpu.CompilerParams(needs_layout_passes=False),
    scratch_types=[
        pltpu.VMEM((NV,), _f32),
        pltpu.VMEM((NV,), _f32),
        pltpu.VMEM((NV,), _f32),
        pltpu.VMEM((FPW,), _i32),
        pltpu.VMEM((FPW,), _i32),
        pltpu.VMEM((FPW,), _i32),
        pltpu.VMEM((NSLOT * CF,), _i32),
        pltpu.VMEM((5 * NSLOT * CF,), _f32),
        pltpu.VMEM((NDIA,), _f32),
        pltpu.SemaphoreType.DMA,
    ],
)

_scatter = pl.kernel(
    _scatter_body,
    out_type=jax.ShapeDtypeStruct((5 * NRANGE * ACC,), _f32),
    mesh=_mesh,
    compiler_params=pltpu.CompilerParams(needs_layout_passes=False),
    scratch_types=[
        pltpu.VMEM((ACC,), _f32),
        pltpu.VMEM((CE,), _i32),
        pltpu.VMEM((CE,), _f32),
        pltpu.VMEM((CE,), _i32),
        pltpu.VMEM((CE,), _f32),
        pltpu.SemaphoreType.DMA,
        pltpu.SemaphoreType.DMA,
    ],
)

_sym = pl.pallas_call(
    _sym_body,
    grid=(5, NV // BM, NV // BM),
    in_specs=[
        pl.BlockSpec((1, BM, BM), lambda m, i, j: (m, i, j)),
        pl.BlockSpec((1, BM, BM), lambda m, i, j: (m, j, i)),
        pl.BlockSpec((1, NW, BM), lambda m, i, j: (m, 0, j)),
    ],
    out_specs=pl.BlockSpec((1, BM, BM), lambda m, i, j: (m, i, j)),
    out_shape=jax.ShapeDtypeStruct((5, NV, NV), _f32),
)


def kernel(vertices, faces):
    vrt = vertices.T.reshape(-1)
    fcs = jnp.pad(faces, ((0, FPAD - NF), (0, 0))).T.reshape(-1)
    idx_e, vals_e, dia = _build(vrt, fcs)
    lower = _scatter(idx_e, vals_e).reshape(5, NV, NV)
    return _sym(lower, lower, dia.reshape(NW, 5, NV).transpose(1, 0, 2))
